# Initial kernel scaffold; baseline (speedup 1.0000x reference)
import jax
import jax.numpy as jnp
from jax.experimental import pallas as pl


def kernel(positions, indeces_i, indeces_j, offsets):
    raise NotImplementedError("write your pallas kernel here")


# speedup vs baseline: 2.1272x; 2.1272x over previous
import functools

import jax
import jax.numpy as jnp
from jax import lax
from jax.experimental import pallas as pl
from jax.experimental.pallas import tpu as pltpu
from jax.experimental.pallas import tpu_sc as plsc

_LANES = 16
_PADW = 16


@functools.lru_cache(maxsize=None)
def _build(N: int, E: int, C: int, interpret: bool):
    try:
        info = plsc.get_sparse_core_info()
        NC, NS = info.num_cores, info.num_subcores
    except ValueError:
        NC, NS = 2, 16
    NW = NC * NS
    assert E % NW == 0, E
    EW = E // NW
    assert EW % C == 0 and C % _LANES == 0
    n_chunks = EW // C
    G = C // _LANES

    mesh = plsc.VectorSubcoreMesh(
        core_axis_name="c", subcore_axis_name="s", num_cores=NC, num_subcores=NS)

    @functools.partial(
        pl.kernel,
        mesh=mesh,
        out_type=jax.ShapeDtypeStruct((3 * E,), jnp.float32),
        scratch_types=[
            pltpu.VMEM((C,), jnp.int32),
            pltpu.VMEM((C,), jnp.int32),
            pltpu.VMEM((C, _PADW), jnp.float32),
            pltpu.VMEM((C, _PADW), jnp.float32),
            pltpu.VMEM((3 * C,), jnp.float32),
            pltpu.SemaphoreType.DMA,
            pltpu.SemaphoreType.DMA,
        ],
        compiler_params=pltpu.CompilerParams(
            use_tc_tiling_on_sc=False, needs_layout_passes=False),
        interpret=interpret,
    )
    def k(pos_hbm, ii_hbm, ij_hbm, off_hbm, out_hbm,
          ii_v, ij_v, pos_i_v, pos_j_v, io_v, sem_i, sem_j):
        wid = lax.axis_index("s") * NC + lax.axis_index("c")
        wbase = wid * EW

        def chunk_body(c, _):
            base = wbase + c * C
            pltpu.sync_copy(ii_hbm.at[pl.ds(base, C)], ii_v)
            pltpu.sync_copy(ij_hbm.at[pl.ds(base, C)], ij_v)
            cp_i = pltpu.async_copy(pos_hbm.at[ii_v], pos_i_v, sem_i)
            cp_j = pltpu.async_copy(pos_hbm.at[ij_v], pos_j_v, sem_j)
            pltpu.sync_copy(off_hbm.at[pl.ds(3 * base, 3 * C)], io_v)
            cp_i.wait()
            cp_j.wait()

            def group_body(g, _):
                e_base = g * _LANES
                ar_in = lax.iota(jnp.int32, _LANES)
                for v in range(3):
                    x = 16 * v + ar_in
                    ediv_c = lax.shift_right_logical(x * 21846, 16)
                    kmod_c = x - 3 * ediv_c
                    e_vec = e_base + ediv_c
                    pi = plsc.load_gather(pos_i_v, [e_vec, kmod_c])
                    pj = plsc.load_gather(pos_j_v, [e_vec, kmod_c])
                    o = g * 48 + 16 * v
                    io_v[pl.ds(o, _LANES)] = pj - pi + io_v[pl.ds(o, _LANES)]
                return 0

            lax.fori_loop(0, G, group_body, 0, unroll=False)
            pltpu.sync_copy(io_v, out_hbm.at[pl.ds(3 * base, 3 * C)])
            return 0

        lax.fori_loop(0, n_chunks, chunk_body, 0, unroll=False)

    return k


def kernel(positions, indeces_i, indeces_j, offsets):
    N, _ = positions.shape
    E = indeces_i.shape[0]
    pos_pad = jnp.pad(positions, ((0, 0), (0, _PADW - positions.shape[1])))
    k = _build(N, E, 2000 if E % (32 * 2000) == 0 else 16, False)
    out_flat = k(
        pos_pad,
        indeces_i.astype(jnp.int32),
        indeces_j.astype(jnp.int32),
        offsets.reshape(-1),
    )
    return out_flat.reshape(E, 3)


# speedup vs baseline: 13.1284x; 6.1717x over previous
import functools

import jax
import jax.numpy as jnp
from jax import lax
from jax.experimental import pallas as pl
from jax.experimental.pallas import tpu as pltpu
from jax.experimental.pallas import tpu_sc as plsc

_LANES = 16
_PADW = 16


@functools.lru_cache(maxsize=None)
def _build(N: int, E: int, C: int, interpret: bool):
    try:
        info = plsc.get_sparse_core_info()
        NC, NS = info.num_cores, info.num_subcores
    except ValueError:
        NC, NS = 2, 16
    NW = NC * NS
    assert E % NW == 0, E
    EW = E // NW
    assert EW % C == 0 and C % _LANES == 0
    n_chunks = EW // C
    G = C // _LANES

    mesh = plsc.VectorSubcoreMesh(
        core_axis_name="c", subcore_axis_name="s", num_cores=NC, num_subcores=NS)

    @functools.partial(
        pl.kernel,
        mesh=mesh,
        out_type=jax.ShapeDtypeStruct((3, E), jnp.float32),
        scratch_types=[
            pltpu.VMEM((C,), jnp.int32),
            pltpu.VMEM((C,), jnp.int32),
            pltpu.VMEM((C, _PADW), jnp.float32),
            pltpu.VMEM((C, _PADW), jnp.float32),
            pltpu.VMEM((3, C), jnp.float32),
            pltpu.SemaphoreType.DMA,
            pltpu.SemaphoreType.DMA,
        ],
        compiler_params=pltpu.CompilerParams(
            use_tc_tiling_on_sc=False, needs_layout_passes=False),
        interpret=interpret,
    )
    def k(pos_hbm, ii_hbm, ij_hbm, off_hbm, out_hbm,
          ii_v, ij_v, pos_i_v, pos_j_v, io_v, sem_i, sem_j):
        wid = lax.axis_index("s") * NC + lax.axis_index("c")
        wbase = wid * EW

        def chunk_body(c, _):
            base = wbase + c * C
            pltpu.sync_copy(ii_hbm.at[pl.ds(base, C)], ii_v)
            pltpu.sync_copy(ij_hbm.at[pl.ds(base, C)], ij_v)
            cp_i = pltpu.async_copy(pos_hbm.at[ii_v], pos_i_v, sem_i)
            cp_j = pltpu.async_copy(pos_hbm.at[ij_v], pos_j_v, sem_j)
            for t in range(3):
                pltpu.sync_copy(off_hbm.at[t, pl.ds(base, C)], io_v.at[t])
            cp_i.wait()
            cp_j.wait()

            def group_body(g, _):
                e_vec = g * _LANES + lax.iota(jnp.int32, _LANES)
                for t in range(3):
                    k_vec = jnp.broadcast_to(jnp.int32(t), (_LANES,))
                    pi = plsc.load_gather(pos_i_v, [e_vec, k_vec])
                    pj = plsc.load_gather(pos_j_v, [e_vec, k_vec])
                    o = g * _LANES
                    io_v[t, pl.ds(o, _LANES)] = (
                        pj - pi + io_v[t, pl.ds(o, _LANES)])
                return 0

            lax.fori_loop(0, G, group_body, 0, unroll=False)
            for t in range(3):
                pltpu.sync_copy(io_v.at[t], out_hbm.at[t, pl.ds(base, C)])
            return 0

        lax.fori_loop(0, n_chunks, chunk_body, 0, unroll=False)

    return k


def kernel(positions, indeces_i, indeces_j, offsets):
    N, _ = positions.shape
    E = indeces_i.shape[0]
    pos_pad = jnp.pad(positions, ((0, 0), (0, _PADW - positions.shape[1])))
    k = _build(N, E, 2000 if E % (32 * 2000) == 0 else 16, False)
    out_t = k(
        pos_pad,
        indeces_i.astype(jnp.int32),
        indeces_j.astype(jnp.int32),
        offsets.T,
    )
    return out_t.T


# speedup vs baseline: 35.4363x; 2.6992x over previous
import functools

import jax
import jax.numpy as jnp
from jax import lax
from jax.experimental import pallas as pl
from jax.experimental.pallas import tpu as pltpu
from jax.experimental.pallas import tpu_sc as plsc

_LANES = 16
_PADW = 16
_TILE = 128
_T = 8


@functools.lru_cache(maxsize=None)
def _build(N: int, E: int, interpret: bool):
    try:
        info = plsc.get_sparse_core_info()
        NC, NS = info.num_cores, info.num_subcores
    except ValueError:
        NC, NS = 2, 16
    NW = NC * NS
    assert NW == 32
    assert E % (_TILE * _T) == 0, E
    NT = E // _TILE
    NCH = NT // _T
    C = _T * _TILE
    M = C // _LANES

    mesh = plsc.VectorSubcoreMesh(
        core_axis_name="c", subcore_axis_name="s", num_cores=NC, num_subcores=NS)

    @functools.partial(
        pl.kernel,
        mesh=mesh,
        out_type=jax.ShapeDtypeStruct((NT, 4, _TILE), jnp.float32),
        scratch_types=[
            pltpu.VMEM((C,), jnp.int32),
            pltpu.VMEM((C,), jnp.int32),
            pltpu.VMEM((C, _PADW), jnp.float32),
            pltpu.VMEM((C, _PADW), jnp.float32),
            pltpu.VMEM((_T, 4, _TILE), jnp.float32),
            pltpu.SemaphoreType.DMA,
            pltpu.SemaphoreType.DMA,
        ],
        compiler_params=pltpu.CompilerParams(
            use_tc_tiling_on_sc=False, needs_layout_passes=False),
        interpret=interpret,
    )
    def k(pos_hbm, ii_hbm, ij_hbm, off_hbm, out_hbm,
          ii_v, ij_v, pos_i_v, pos_j_v, io_v, sem_i, sem_j):
        wid = lax.axis_index("s") * NC + lax.axis_index("c")
        c_lo = lax.shift_right_logical(wid * NCH, 5)
        c_hi = lax.shift_right_logical((wid + 1) * NCH, 5)

        def chunk_body(c, _):
            base = c * C
            t0 = c * _T
            pltpu.sync_copy(ii_hbm.at[pl.ds(base, C)], ii_v)
            pltpu.sync_copy(ij_hbm.at[pl.ds(base, C)], ij_v)
            cp_i = pltpu.async_copy(pos_hbm.at[ii_v], pos_i_v, sem_i)
            cp_j = pltpu.async_copy(pos_hbm.at[ij_v], pos_j_v, sem_j)
            pltpu.sync_copy(off_hbm.at[pl.ds(t0, _T)], io_v)
            cp_i.wait()
            cp_j.wait()

            def group_body(m, _):
                tt = lax.shift_right_logical(m, 3)
                lo = 16 * (m & 7)
                e_vec = m * _LANES + lax.iota(jnp.int32, _LANES)
                for s in range(3):
                    k_vec = jnp.broadcast_to(jnp.int32(s), (_LANES,))
                    pi = plsc.load_gather(pos_i_v, [e_vec, k_vec])
                    pj = plsc.load_gather(pos_j_v, [e_vec, k_vec])
                    io_v[tt, s, pl.ds(lo, _LANES)] = (
                        pj - pi + io_v[tt, s, pl.ds(lo, _LANES)])
                return 0

            lax.fori_loop(0, M, group_body, 0, unroll=False)
            pltpu.sync_copy(io_v, out_hbm.at[pl.ds(t0, _T)])
            return 0

        lax.fori_loop(c_lo, c_hi, chunk_body, 0, unroll=False)

    return k


def kernel(positions, indeces_i, indeces_j, offsets):
    N, _ = positions.shape
    E = indeces_i.shape[0]
    pos_pad = jnp.pad(positions, ((0, 0), (0, _PADW - positions.shape[1])))
    off_t = (jnp.pad(offsets, ((0, 0), (0, 1)))
             .reshape(E // _TILE, _TILE, 4)
             .transpose(0, 2, 1))
    k = _build(N, E, False)
    out_t = k(
        pos_pad,
        indeces_i.astype(jnp.int32),
        indeces_j.astype(jnp.int32),
        off_t,
    )
    return out_t.transpose(0, 2, 1).reshape(E, 4)[:, :3]


# speedup vs baseline: 44.6032x; 1.2587x over previous
import functools

import jax
import jax.numpy as jnp
from jax import lax
from jax.experimental import pallas as pl
from jax.experimental.pallas import tpu as pltpu
from jax.experimental.pallas import tpu_sc as plsc

_LANES = 16
_PADW = 8
_TILE = 128
_T = 20


@functools.lru_cache(maxsize=None)
def _build(N: int, E: int, interpret: bool):
    try:
        info = plsc.get_sparse_core_info()
        NC, NS = info.num_cores, info.num_subcores
    except ValueError:
        NC, NS = 2, 16
    NW = NC * NS
    assert NW == 32
    assert E % (_TILE * _T) == 0, E
    NT = E // _TILE
    NCH = NT // _T
    C = _T * _TILE
    M = C // _LANES

    mesh = plsc.VectorSubcoreMesh(
        core_axis_name="c", subcore_axis_name="s", num_cores=NC, num_subcores=NS)

    @functools.partial(
        pl.kernel,
        mesh=mesh,
        out_type=jax.ShapeDtypeStruct((NT, 4, _TILE), jnp.float32),
        scratch_types=[
            pltpu.VMEM((C,), jnp.int32),
            pltpu.VMEM((C,), jnp.int32),
            pltpu.VMEM((C, _PADW), jnp.float32),
            pltpu.VMEM((C, _PADW), jnp.float32),
            pltpu.VMEM((_T, 4, _TILE), jnp.float32),
            pltpu.SemaphoreType.DMA,
            pltpu.SemaphoreType.DMA,
        ],
        compiler_params=pltpu.CompilerParams(
            use_tc_tiling_on_sc=False, needs_layout_passes=False),
        interpret=interpret,
    )
    def k(pos_hbm, ii_hbm, ij_hbm, off_hbm, out_hbm,
          ii_v, ij_v, pos_i_v, pos_j_v, io_v, sem_i, sem_j):
        wid = lax.axis_index("s") * NC + lax.axis_index("c")
        c_lo = lax.shift_right_logical(wid * NCH, 5)
        c_hi = lax.shift_right_logical((wid + 1) * NCH, 5)

        def chunk_body(c, _):
            base = c * C
            t0 = c * _T
            pltpu.sync_copy(ii_hbm.at[pl.ds(base, C)], ii_v)
            pltpu.sync_copy(ij_hbm.at[pl.ds(base, C)], ij_v)
            cp_i = pltpu.async_copy(pos_hbm.at[ii_v], pos_i_v, sem_i)
            cp_j = pltpu.async_copy(pos_hbm.at[ij_v], pos_j_v, sem_j)
            pltpu.sync_copy(off_hbm.at[pl.ds(t0, _T)], io_v)
            cp_i.wait()
            cp_j.wait()

            def group_body(m, _):
                tt = lax.shift_right_logical(m, 3)
                lo = 16 * (m & 7)
                e_vec = m * _LANES + lax.iota(jnp.int32, _LANES)
                for s in range(3):
                    k_vec = jnp.broadcast_to(jnp.int32(s), (_LANES,))
                    pi = plsc.load_gather(pos_i_v, [e_vec, k_vec])
                    pj = plsc.load_gather(pos_j_v, [e_vec, k_vec])
                    io_v[tt, s, pl.ds(lo, _LANES)] = (
                        pj - pi + io_v[tt, s, pl.ds(lo, _LANES)])
                return 0

            lax.fori_loop(0, M, group_body, 0, unroll=False)
            pltpu.sync_copy(io_v, out_hbm.at[pl.ds(t0, _T)])
            return 0

        lax.fori_loop(c_lo, c_hi, chunk_body, 0, unroll=False)

    return k


def kernel(positions, indeces_i, indeces_j, offsets):
    N, _ = positions.shape
    E = indeces_i.shape[0]
    pos_pad = jnp.pad(positions, ((0, 0), (0, _PADW - positions.shape[1])))
    off_t = (jnp.pad(offsets, ((0, 0), (0, 1)))
             .reshape(E // _TILE, _TILE, 4)
             .transpose(0, 2, 1))
    k = _build(N, E, False)
    out_t = k(
        pos_pad,
        indeces_i.astype(jnp.int32),
        indeces_j.astype(jnp.int32),
        off_t,
    )
    return out_t.transpose(0, 2, 1).reshape(E, 4)[:, :3]
